# baseline (speedup 1.0000x reference)
import functools

import jax
import jax.numpy as jnp
from jax.experimental import pallas as pl
from jax.experimental.pallas import tpu as pltpu


def _diag_kernel(x_ref, idx_ref, o_ref, *, O, J, B):
    n_loc = O * J
    w = 4 * O * J
    x = x_ref[:, 0, :, 0, 0, :]

    riota = jax.lax.broadcasted_iota(jnp.int32, (w, n_loc), 0)

    def gsel(cls, oj):
        a = x.reshape(B, O, 2 * J, w)[:, :, J * oj:J * oj + J, :]
        xr = a.reshape(B * n_loc, w)
        sel = jnp.where(riota == idx_ref[cls, 0], 1.0, 0.0)
        return jax.lax.dot(xr, sel, preferred_element_type=jnp.float32)

    g_b = gsel(0, 0)
    g_c = gsel(1, 1)
    mrow = jax.lax.broadcasted_iota(jnp.int32, (B * n_loc, n_loc), 0)
    ncol = jax.lax.broadcasted_iota(jnp.int32, (B * n_loc, n_loc), 1)
    d_j_loc = ((mrow % n_loc) // J) == (ncol // J)
    contrib = jnp.where(d_j_loc, g_b, 0.0) + g_c
    o_ref[:, 0] = contrib.reshape(B, n_loc, n_loc)


def _main_kernel(x_ref, bc_ref, idx_ref, o_ref, *, O, J):
    n_in = x_ref.shape[4]
    n_out = o_ref.shape[1]
    x = x_ref[0, :, 0]

    row = jax.lax.broadcasted_iota(jnp.int32, (n_out, n_out), 0)
    col = jax.lax.broadcasted_iota(jnp.int32, (n_out, n_out), 1)
    d_j = ((row // J) % O) == ((col // J) % O)

    riota = jax.lax.broadcasted_iota(jnp.int32, (n_in, n_out), 0)

    def gsel(cls, oj):
        a = x.reshape(O, O, 2 * J, n_in)[:, :, J * oj:J * oj + J, :]
        xr = a.reshape(n_out, n_in)
        sel = jnp.where(riota == idx_ref[cls], 1.0, 0.0)
        return jax.lax.dot(xr, sel, preferred_element_type=jnp.float32)

    g_a = gsel(0, 1)
    g_d = gsel(1, 0)
    o_ref[0] = g_a + jnp.where(d_j, g_d, 0.0)
    bc = bc_ref[0]
    n_loc = O * J
    for i in range(O):
        blk = o_ref[0, n_loc * i:n_loc * (i + 1), n_loc * i:n_loc * (i + 1)]
        o_ref[0, n_loc * i:n_loc * (i + 1), n_loc * i:n_loc * (i + 1)] = (
            blk + bc[i])


def kernel(input, cols, mask):
    B, n_in, _ = input.shape
    K, n_out = cols.shape
    O = int(round(K ** 0.5)) - 1
    J = n_out // (O * O)
    n_loc = O * J
    sb = 4 * O * J

    safe = jnp.where(mask, cols, 0).astype(jnp.int32)
    idx_a = safe[0]
    idx_b = jnp.sum(safe[1:1 + O * O], axis=0)
    idx_c = jnp.sum(safe[1 + O * O:1 + O * O + O], axis=0)
    idx_d = jnp.sum(safe[1 + O * O + O:], axis=0)
    off = sb * jnp.arange(O, dtype=jnp.int32)[None, :, None]
    idx_bc = (jnp.stack([idx_b, idx_c]).reshape(2, O, n_loc) - off)
    idx_bc = idx_bc.reshape(2, O, 1, n_loc)
    idx_ad = jnp.stack([idx_a, idx_d]).reshape(2, 1, n_out)

    x_diag_view = input.reshape(B, 2 * O, 2 * O * J, O, 1, sb)
    bc = pl.pallas_call(
        functools.partial(_diag_kernel, O=O, J=J, B=B),
        grid=(O,),
        in_specs=[
            pl.BlockSpec((B, 1, 2 * O * J, 1, 1, sb),
                         lambda i: (0, 2 * i, 0, i, 0, 0)),
            pl.BlockSpec((2, 1, 1, n_loc), lambda i: (0, i, 0, 0)),
        ],
        out_specs=pl.BlockSpec((B, 1, n_loc, n_loc), lambda i: (0, i, 0, 0)),
        out_shape=jax.ShapeDtypeStruct((B, O, n_loc, n_loc), jnp.float32),
        compiler_params=pltpu.CompilerParams(
            dimension_semantics=("arbitrary",),
        ),
        name="pool2d_diag_bc",
    )(x_diag_view, idx_bc)

    x_slab_view = input.reshape(B, O, 2, 2 * O * J, n_in)
    fn = pl.pallas_call(
        functools.partial(_main_kernel, O=O, J=J),
        grid=(B,),
        in_specs=[
            pl.BlockSpec((1, O, 1, 2 * O * J, n_in),
                         lambda b: (b, 0, 1, 0, 0)),
            pl.BlockSpec((1, O, n_loc, n_loc), lambda b: (b, 0, 0, 0)),
            pl.BlockSpec((2, 1, n_out), lambda b: (0, 0, 0)),
        ],
        out_specs=pl.BlockSpec((1, n_out, n_out), lambda b: (b, 0, 0)),
        out_shape=jax.ShapeDtypeStruct((B, n_out, n_out), jnp.float32),
        compiler_params=pltpu.CompilerParams(
            dimension_semantics=("parallel",),
        ),
        name="pool2d_main_ad",
    )
    return fn(x_slab_view, bc, idx_ad)
